# baseline (speedup 1.0000x reference)
import functools

import jax
import jax.numpy as jnp
from jax import lax
from jax.experimental import pallas as pl
from jax.experimental.pallas import tpu as pltpu
from jax.experimental.pallas import tpu_sc as plsc

_SEQ_BLK = 1024


def _add_kernel(x_ref, t_ref, o_ref):
    o_ref[...] = x_ref[...] + t_ref[...]


def _tc_impl(inputs, pos_table):
    B, S, D = inputs.shape
    ns = S // _SEQ_BLK
    return pl.pallas_call(
        _add_kernel,
        grid=(ns, B),
        in_specs=[
            pl.BlockSpec((1, _SEQ_BLK, D), lambda s, b: (b, s, 0)),
            pl.BlockSpec((_SEQ_BLK, D), lambda s, b: (s, 0)),
        ],
        out_specs=pl.BlockSpec((1, _SEQ_BLK, D), lambda s, b: (b, s, 0)),
        out_shape=jax.ShapeDtypeStruct(inputs.shape, inputs.dtype),
        compiler_params=pltpu.CompilerParams(
            dimension_semantics=("parallel", "arbitrary"),
        ),
    )(inputs, pos_table)


_NW = 32
_C = 8
_XRING = 5


def _sc_impl(x1d, t1d, S, D):
    R = x1d.shape[0] // D
    NB = R // S
    span = S // _NW
    T = span // _C
    J = T * NB
    CW = _C * D

    mesh = plsc.VectorSubcoreMesh(core_axis_name="c", subcore_axis_name="s")

    @functools.partial(
        pl.kernel,
        mesh=mesh,
        out_type=jax.ShapeDtypeStruct((R * D,), jnp.float32),
        scratch_types=[
            pltpu.VMEM((2 * CW,), jnp.float32),
            pltpu.VMEM((_XRING * CW,), jnp.float32),
            pltpu.SemaphoreType.DMA,
            pltpu.SemaphoreType.DMA,
            pltpu.SemaphoreType.DMA,
        ],
    )
    def sc_k(x_hbm, t_hbm, o_hbm, tbuf, xbuf, tsem, xsem, osem):
        wid = lax.axis_index("c") * 16 + lax.axis_index("s")
        p0 = wid * span

        def t_copy(tau):
            slot = lax.rem(tau, 2) * CW
            return pltpu.make_async_copy(
                t_hbm.at[pl.ds((p0 + tau * _C) * D, CW)],
                tbuf.at[pl.ds(slot, CW)],
                tsem,
            )

        def x_off(j):
            tau = lax.div(j, NB)
            b = lax.rem(j, NB)
            return (b * S + p0 + tau * _C) * D

        def x_slot(j):
            return lax.rem(j, _XRING) * CW

        def x_copy(j):
            return pltpu.make_async_copy(
                x_hbm.at[pl.ds(x_off(j), CW)],
                xbuf.at[pl.ds(x_slot(j), CW)],
                xsem,
            )

        def o_copy(j):
            return pltpu.make_async_copy(
                xbuf.at[pl.ds(x_slot(j), CW)],
                o_hbm.at[pl.ds(x_off(j), CW)],
                osem,
            )

        t_copy(0).start()
        t_copy(1).start()
        x_copy(0).start()
        x_copy(1).start()

        def body(j, carry):
            tau = lax.div(j, NB)
            b = lax.rem(j, NB)

            @pl.when(b == 0)
            def _twait():
                t_copy(tau).wait()

            x_copy(j).wait()
            tbase = lax.rem(tau, 2) * CW
            xbase = x_slot(j)

            @plsc.parallel_loop(0, CW // 16, unroll=8)
            def add_body(i, _xbase=xbase, _tbase=tbase):
                off = i * 16
                tv = tbuf[pl.ds(_tbase + off, 16)]
                plsc.addupdate(xbuf.at[pl.ds(_xbase + off, 16)], tv)

            o_copy(j).start()

            @pl.when((b == NB - 1) & (tau + 2 < T))
            def _tprefetch():
                t_copy(tau + 2).start()

            @pl.when(j + 2 < J)
            def _xprefetch():
                @pl.when(j >= _XRING - 2)
                def _drain():
                    o_copy(j - (_XRING - 2)).wait()

                x_copy(j + 2).start()

            return carry

        lax.fori_loop(0, J, body, 0)
        for j in range(J - _XRING, J):
            o_copy(j).wait()

    return sc_k(x1d, t1d)


def kernel(inputs, pos_table):
    B, S, D = inputs.shape
    out = _sc_impl(inputs.reshape(-1), pos_table.reshape(-1), S, D)
    return out.reshape(B, S, D)


# speedup vs baseline: 4.1670x; 4.1670x over previous
import functools

import jax
import jax.numpy as jnp
from jax import lax
from jax.experimental import pallas as pl
from jax.experimental.pallas import tpu as pltpu
from jax.experimental.pallas import tpu_sc as plsc

_SEQ_BLK = 1024


def _add_kernel(x_ref, t_ref, o_ref):
    o_ref[...] = x_ref[...] + t_ref[...]


def _tc_impl(inputs, pos_table):
    B, S, D = inputs.shape
    ns = S // _SEQ_BLK
    return pl.pallas_call(
        _add_kernel,
        grid=(ns, B),
        in_specs=[
            pl.BlockSpec((1, _SEQ_BLK, D), lambda s, b: (b, s, 0)),
            pl.BlockSpec((_SEQ_BLK, D), lambda s, b: (s, 0)),
        ],
        out_specs=pl.BlockSpec((1, _SEQ_BLK, D), lambda s, b: (b, s, 0)),
        out_shape=jax.ShapeDtypeStruct(inputs.shape, inputs.dtype),
        compiler_params=pltpu.CompilerParams(
            dimension_semantics=("parallel", "arbitrary"),
        ),
    )(inputs, pos_table)


_NW = 32
_C = 4


def _sc_impl(x1d, t1d, S, D):
    R = x1d.shape[0] // D
    NB = R // S
    span = S // _NW
    T = span // _C
    CW = _C * D

    mesh = plsc.VectorSubcoreMesh(core_axis_name="c", subcore_axis_name="s")

    @functools.partial(
        pl.kernel,
        mesh=mesh,
        out_type=jax.ShapeDtypeStruct((R * D,), jnp.float32),
        scratch_types=[
            pltpu.VMEM((2 * CW,), jnp.float32),
            pltpu.VMEM((3 * NB * CW,), jnp.float32),
            pltpu.SemaphoreType.DMA,
            pltpu.SemaphoreType.DMA,
            pltpu.SemaphoreType.DMA,
        ],
    )
    def sc_k(x_hbm, t_hbm, o_hbm, tbuf, xbuf, tsem, xsem, osem):
        wid = lax.axis_index("c") * 16 + lax.axis_index("s")
        p0 = wid * span

        def t_copy(tau):
            slot = lax.rem(tau, 2) * CW
            return pltpu.make_async_copy(
                t_hbm.at[pl.ds((p0 + tau * _C) * D, CW)],
                tbuf.at[pl.ds(slot, CW)],
                tsem,
            )

        def x_off(tau, b):
            return (b * S + p0 + tau * _C) * D

        def x_slot(tau, b):
            return (lax.rem(tau, 3) * NB + b) * CW

        def x_copy(tau, b):
            return pltpu.make_async_copy(
                x_hbm.at[pl.ds(x_off(tau, b), CW)],
                xbuf.at[pl.ds(x_slot(tau, b), CW)],
                xsem,
            )

        def o_copy(tau, b):
            return pltpu.make_async_copy(
                xbuf.at[pl.ds(x_slot(tau, b), CW)],
                o_hbm.at[pl.ds(x_off(tau, b), CW)],
                osem,
            )

        t_copy(0).start()
        for b in range(NB):
            x_copy(0, b).start()
        t_copy(1).start()
        for b in range(NB):
            x_copy(1, b).start()

        def body(tau, carry):
            t_copy(tau).wait()
            tbase = lax.rem(tau, 2) * CW
            for b in range(NB):
                x_copy(tau, b).wait()
                xbase = x_slot(tau, b)

                @plsc.parallel_loop(0, CW // 16, unroll=8)
                def add_body(i, _xbase=xbase, _tbase=tbase):
                    off = pl.multiple_of(i * 16, 16)
                    tv = tbuf[pl.ds(pl.multiple_of(_tbase + off, 16), 16)]
                    plsc.addupdate(
                        xbuf.at[pl.ds(pl.multiple_of(_xbase + off, 16), 16)], tv
                    )

                o_copy(tau, b).start()

            @pl.when(tau + 2 < T)
            def _prefetch():
                @pl.when(tau >= 1)
                def _drain():
                    for b in range(NB):
                        o_copy(tau - 1, b).wait()

                t_copy(tau + 2).start()
                for b in range(NB):
                    x_copy(tau + 2, b).start()

            return carry

        lax.fori_loop(0, T, body, 0)
        for tau in (T - 3, T - 2, T - 1):
            for b in range(NB):
                o_copy(tau, b).wait()

    return sc_k(x1d, t1d)


def kernel(inputs, pos_table):
    return _tc_impl(inputs, pos_table)
